# Initial kernel scaffold; baseline (speedup 1.0000x reference)
import jax
import jax.numpy as jnp
from jax.experimental import pallas as pl


def kernel(center, pos, neg, input_table, output_table):
    raise NotImplementedError("write your pallas kernel here")


# speedup vs baseline: 3.9661x; 3.9661x over previous
import functools

import jax
import jax.numpy as jnp
from jax import lax
from jax.experimental import pallas as pl
from jax.experimental.pallas import tpu as pltpu
from jax.experimental.pallas import tpu_sc as plsc

V = 1000000
D = 64
B = 16384
NNEG = 20

NC = 2
NS = 16
L = 16
NW = NC * NS
NB = B // NW
C = 32
NCHUNK = NB // C
NEGC = C * NNEG
ROWS = NNEG + 1
ROWS_PAD = 24


def _sc_body(center_hbm, pos_hbm, neg_hbm, in_tab, out_tab, scores_hbm,
             cen_idx, pos_idx, neg_idx, cen_rows, pos_rows, neg_rows,
             scores_v, sem):
    wid = lax.axis_index("s") * NC + lax.axis_index("c")
    base = pl.multiple_of(wid * NB, NB)

    pltpu.sync_copy(center_hbm.at[pl.ds(base, NB)], cen_idx)
    pltpu.sync_copy(pos_hbm.at[pl.ds(base, NB)], pos_idx)
    pltpu.sync_copy(neg_hbm.at[pl.ds(base * NNEG, NB * NNEG)], neg_idx)

    lane = lax.iota(jnp.int32, L)

    def chunk_body(g, carry):
        goff = pl.multiple_of(g * C, C)
        copies = [
            pltpu.async_copy(in_tab.at[cen_idx.at[pl.ds(goff, C)]],
                             cen_rows, sem),
            pltpu.async_copy(out_tab.at[pos_idx.at[pl.ds(goff, C)]],
                             pos_rows, sem),
        ]
        for k in range(NEGC // 128):
            copies.append(pltpu.async_copy(
                out_tab.at[neg_idx.at[pl.ds(goff * NNEG + k * 128, 128)]],
                neg_rows.at[pl.ds(k * 128, 128)], sem))
        for cp in copies:
            cp.wait()

        for t in range(C // L):
            r = t * L + lane
            rn = [r * NNEG + n for n in range(NNEG)]

            def d_body(dd, accs):
                dv = jnp.broadcast_to(dd, (L,)).astype(jnp.int32)
                cen_d = plsc.load_gather(cen_rows, [r, dv])
                pos_d = plsc.load_gather(pos_rows, [r, dv])
                new = [accs[0] + cen_d * pos_d]
                for n in range(NNEG):
                    neg_d = plsc.load_gather(neg_rows, [rn[n], dv])
                    new.append(accs[n + 1] + cen_d * neg_d)
                return tuple(new)

            accs = lax.fori_loop(
                0, D, d_body,
                tuple(jnp.zeros((L,), jnp.float32) for _ in range(ROWS)))
            col = goff + t * L
            scores_v[0, pl.ds(col, L)] = accs[0]
            for n in range(NNEG):
                scores_v[1 + n, pl.ds(col, L)] = -accs[1 + n]
        return carry

    lax.fori_loop(0, NCHUNK, chunk_body, 0)

    pad = jnp.full((L,), 40.0, jnp.float32)
    for j in range(ROWS, ROWS_PAD):
        for c0 in range(0, NB, L):
            scores_v[j, pl.ds(c0, L)] = pad

    pltpu.sync_copy(scores_v, scores_hbm.at[:, pl.ds(base, NB)])


_sc_scores = functools.partial(
    pl.kernel,
    out_type=jax.ShapeDtypeStruct((ROWS_PAD, B), jnp.float32),
    mesh=plsc.VectorSubcoreMesh(core_axis_name="c", subcore_axis_name="s"),
    scratch_types=[
        pltpu.VMEM((NB,), jnp.int32),
        pltpu.VMEM((NB,), jnp.int32),
        pltpu.VMEM((NB * NNEG,), jnp.int32),
        pltpu.VMEM((C, D), jnp.float32),
        pltpu.VMEM((C, D), jnp.float32),
        pltpu.VMEM((NEGC, D), jnp.float32),
        pltpu.VMEM((ROWS_PAD, NB), jnp.float32),
        pltpu.SemaphoreType.DMA,
    ],
    compiler_params=pltpu.CompilerParams(needs_layout_passes=False,
                                         use_tc_tiling_on_sc=False),
)(_sc_body)


def _tc_loss_body(scores_ref, out_ref):
    x = scores_ref[...]
    row = lax.broadcasted_iota(jnp.int32, x.shape, 0)
    val = -jnp.log(jax.nn.sigmoid(x) + 1e-10)
    out_ref[0, 0] = jnp.sum(jnp.where(row < ROWS, val, 0.0)) / B


_tc_loss = pl.pallas_call(
    _tc_loss_body,
    out_shape=jax.ShapeDtypeStruct((1, 1), jnp.float32),
    in_specs=[pl.BlockSpec(memory_space=pltpu.VMEM)],
    out_specs=pl.BlockSpec(memory_space=pltpu.SMEM),
)


def kernel(center, pos, neg, input_table, output_table):
    scores = _sc_scores(center.astype(jnp.int32), pos.astype(jnp.int32),
                        neg.reshape(-1).astype(jnp.int32),
                        input_table, output_table)
    return _tc_loss(scores)[0, 0]
